# baseline (speedup 1.0000x reference)
import functools
import math

import jax
import jax.numpy as jnp
from jax import lax
from jax.experimental import pallas as pl
from jax.experimental.pallas import tpu as pltpu
from jax.experimental.pallas import tpu_sc as plsc

_EMBED = 128
_LANES = 16
_CHUNK = 128


def _make_pe_table(seq_len: int) -> jax.Array:
    position = jnp.arange(seq_len, dtype=jnp.float32)[:, None]
    div_term = jnp.exp(
        jnp.arange(0, _EMBED, 2, dtype=jnp.float32) * (-math.log(10000.0) / _EMBED)
    )
    ang = position * div_term
    pe = jnp.zeros((seq_len, _EMBED), dtype=jnp.float32)
    pe = pe.at[:, 0::2].set(jnp.sin(ang))
    pe = pe.at[:, 1::2].set(jnp.cos(ang))
    return pe


@functools.partial(jax.jit, static_argnames=("n_rows", "seq_len"))
def _gather_add_pe(table, idx2d, pe, *, n_rows, seq_len):
    info = plsc.get_sparse_core_info()
    nc, ns = info.num_cores, info.num_subcores
    nw = nc * ns
    rows_per_w = n_rows // nw
    n_chunks = rows_per_w // _CHUNK
    chunks_per_seq = seq_len // _CHUNK

    mesh = plsc.VectorSubcoreMesh(core_axis_name="c", subcore_axis_name="s")

    @functools.partial(
        pl.kernel,
        out_type=jax.ShapeDtypeStruct((n_rows, _EMBED), jnp.float32),
        mesh=mesh,
        scratch_types=[
            pltpu.VMEM((n_chunks, _CHUNK), jnp.int32),
            pltpu.VMEM((_CHUNK, _EMBED), jnp.float32),
            pltpu.VMEM((_CHUNK, _EMBED), jnp.float32),
            pltpu.SemaphoreType.DMA,
        ],
    )
    def body(table_hbm, idx_hbm, pe_hbm, out_hbm, idx_v, rows_v, pe_v, sem):
        wid = lax.axis_index("s") * nc + lax.axis_index("c")
        base_chunk = wid * n_chunks
        pltpu.sync_copy(idx_hbm.at[pl.ds(base_chunk, n_chunks)], idx_v)
        for j in range(n_chunks):
            chunk = base_chunk + j
            pos_chunk = lax.rem(chunk, chunks_per_seq)
            pltpu.async_copy(table_hbm.at[idx_v.at[j]], rows_v, sem).wait()
            pltpu.sync_copy(pe_hbm.at[pl.ds(pos_chunk * _CHUNK, _CHUNK)], pe_v)

            def add_row(i, _):
                for c in range(_EMBED // _LANES):
                    sl = pl.ds(c * _LANES, _LANES)
                    rows_v[i, sl] = rows_v[i, sl] + pe_v[i, sl]
                return 0

            lax.fori_loop(0, _CHUNK, add_row, 0)
            pltpu.sync_copy(rows_v, out_hbm.at[pl.ds(chunk * _CHUNK, _CHUNK)])

    return body(table, idx2d, pe)


def kernel(x, table):
    batch, seq = x.shape
    n_rows = batch * seq
    idx2d = x.reshape(n_rows // _CHUNK, _CHUNK)
    pe = _make_pe_table(seq)
    out = _gather_add_pe(table, idx2d, pe, n_rows=n_rows, seq_len=seq)
    return out.reshape(batch, seq, _EMBED)
